# baseline (speedup 1.0000x reference)
import jax
import jax.numpy as jnp
from jax.experimental import pallas as pl
from jax.experimental.pallas import tpu as pltpu

_NCHUNK = 8


def _dma_copy_body(x_ref, o_ref, sems):
    for i in range(_NCHUNK):
        pltpu.make_async_copy(x_ref.at[i], o_ref.at[i], sems.at[i]).start()
    for i in range(_NCHUNK):
        pltpu.make_async_copy(x_ref.at[i], o_ref.at[i], sems.at[i]).wait()


def _dma_copy(x2):
    return pl.pallas_call(
        _dma_copy_body,
        in_specs=[pl.BlockSpec(memory_space=pl.ANY)],
        out_specs=pl.BlockSpec(memory_space=pl.ANY),
        out_shape=jax.ShapeDtypeStruct(x2.shape, x2.dtype),
        scratch_shapes=[pltpu.SemaphoreType.DMA((_NCHUNK,))],
    )(x2)


def _add_body(s_ref, x_ref, o_ref):
    o_ref[...] = x_ref[...] + s_ref[0]


def _blocked_add(x2, s):
    rows, lanes = x2.shape
    bm = 128
    return pl.pallas_call(
        _add_body,
        grid=(rows // bm,),
        in_specs=[
            pl.BlockSpec(memory_space=pltpu.SMEM),
            pl.BlockSpec((bm, lanes), lambda i: (i, 0)),
        ],
        out_specs=pl.BlockSpec((bm, lanes), lambda i: (i, 0)),
        out_shape=jax.ShapeDtypeStruct((rows, lanes), x2.dtype),
        compiler_params=pltpu.CompilerParams(
            dimension_semantics=("arbitrary",),
        ),
    )(s.reshape(1), x2)


def kernel(input, pad, channels_last):
    x = input
    s = jnp.asarray(pad, x.dtype) + jnp.asarray(channels_last, x.dtype)
    lanes = x.shape[-1] * x.shape[-2]
    rows = x.size // lanes
    x3 = x.reshape(_NCHUNK, rows // _NCHUNK, lanes)
    out = jax.lax.cond(
        s == 0,
        lambda v: _dma_copy(v),
        lambda v: _blocked_add(
            v.reshape(rows, lanes), s).reshape(v.shape),
        x3,
    )
    return out.reshape(x.shape)


# speedup vs baseline: 15.3467x; 15.3467x over previous
import jax
import jax.numpy as jnp
from jax.experimental import pallas as pl
from jax.experimental.pallas import tpu as pltpu

_LANES = 16384
_ROWS = 3072
_BM = 64
_K = 4
_N = _ROWS // _BM


def _pipe_body(s_ref, x_hbm, o_hbm, xbuf, obuf, insem, outsem):
    def in_copy(t, slot):
        return pltpu.make_async_copy(
            x_hbm.at[pl.ds(t * _BM, _BM)], xbuf.at[slot], insem.at[slot])

    def out_copy(t, slot):
        return pltpu.make_async_copy(
            obuf.at[slot], o_hbm.at[pl.ds(t * _BM, _BM)], outsem.at[slot])

    for t in range(_K):
        in_copy(t, t).start()
    for t in range(_N):
        slot = t % _K
        in_copy(t, slot).wait()
        if t >= _K:
            out_copy(t - _K, slot).wait()
        obuf[slot] = xbuf[slot] + s_ref[0]
        out_copy(t, slot).start()
        if t + _K < _N:
            in_copy(t + _K, slot).start()
    for t in range(_N - _K, _N):
        out_copy(t, t % _K).wait()


def kernel(input, pad, channels_last):
    x = input
    s = (jnp.asarray(pad, x.dtype) + jnp.asarray(channels_last, x.dtype)).reshape(1)
    x2 = x.reshape(_ROWS, _LANES)
    out = pl.pallas_call(
        _pipe_body,
        in_specs=[
            pl.BlockSpec(memory_space=pltpu.SMEM),
            pl.BlockSpec(memory_space=pl.ANY),
        ],
        out_specs=pl.BlockSpec(memory_space=pl.ANY),
        out_shape=jax.ShapeDtypeStruct((_ROWS, _LANES), x.dtype),
        scratch_shapes=[
            pltpu.VMEM((_K, _BM, _LANES), x.dtype),
            pltpu.VMEM((_K, _BM, _LANES), x.dtype),
            pltpu.SemaphoreType.DMA((_K,)),
            pltpu.SemaphoreType.DMA((_K,)),
        ],
    )(s, x2)
    return out.reshape(x.shape)


# speedup vs baseline: 50.7606x; 3.3076x over previous
import jax
import jax.numpy as jnp
from jax.experimental import pallas as pl
from jax.experimental.pallas import tpu as pltpu


def _add_body(s_ref, x_ref, o_ref):
    o_ref[...] = x_ref[...] + s_ref[0]


def kernel(input, pad, channels_last):
    x = input
    s = (jnp.asarray(pad, x.dtype) + jnp.asarray(channels_last, x.dtype)).reshape(1)
    lanes = x.shape[-1]
    rows = x.size // lanes
    bm = 8192
    while rows % bm:
        bm //= 2
    x2 = x.reshape(rows, lanes)
    out = pl.pallas_call(
        _add_body,
        grid=(rows // bm,),
        in_specs=[
            pl.BlockSpec(memory_space=pltpu.SMEM),
            pl.BlockSpec((bm, lanes), lambda i: (i, 0)),
        ],
        out_specs=pl.BlockSpec((bm, lanes), lambda i: (i, 0)),
        out_shape=jax.ShapeDtypeStruct((rows, lanes), x.dtype),
        compiler_params=pltpu.CompilerParams(
            dimension_semantics=("arbitrary",),
        ),
    )(s, x2)
    return out.reshape(x.shape)
